# baseline (speedup 1.0000x reference)
import jax
import jax.numpy as jnp
from jax.experimental import pallas as pl

N = 10000
RB = 1000


def _ln(x, g, b, eps=1e-5):
    mu = jnp.mean(x, axis=-1, keepdims=True)
    var = jnp.mean((x - mu) ** 2, axis=-1, keepdims=True)
    return (x - mu) / jnp.sqrt(var + eps) * g + b


def _sage(x, edge_index, Wl, bl, Wr, n):
    src = edge_index[0]
    dst = edge_index[1]
    agg = jax.ops.segment_sum(x[src], dst, num_segments=n)
    cnt = jax.ops.segment_sum(jnp.ones((edge_index.shape[1],), x.dtype), dst, num_segments=n)
    mean = agg / jnp.clip(cnt, 1.0)[:, None]
    return mean @ Wl.T + bl + x @ Wr.T


def _final_block(x_ref, y_ref, o_ref):
    o_ref[...] = jax.lax.dot_general(
        x_ref[...], y_ref[...], (((1,), (1,)), ((), ())),
        preferred_element_type=jnp.float32)


def _final_matmul(x, y):
    RBF = 400
    return pl.pallas_call(
        _final_block,
        grid=(N // RBF,),
        in_specs=[pl.BlockSpec((RBF, 64), lambda i: (i, 0)),
                  pl.BlockSpec((N, 64), lambda i: (0, 0))],
        out_specs=pl.BlockSpec((RBF, N), lambda i: (i, 0)),
        out_shape=jax.ShapeDtypeStruct((N, N), jnp.float32),
    )(x, y)


def kernel(mm_edge_index, dd_edge_index, x_m, x_d,
           Wl_x1, bl_x1, Wr_x1, g_x1, b_x1,
           Wl_x2, bl_x2, Wr_x2, g_x2, b_x2,
           Wl_y1, bl_y1, Wr_y1, g_y1, b_y1,
           Wl_y2, bl_y2, Wr_y2, g_y2, b_y2,
           W1x, b1x, W2x, b2x, W3x, b3x,
           W1y, b1y, W2y, b2y, W3y, b3y):
    M = x_m.shape[0]
    D = x_d.shape[0]
    X1 = jax.nn.relu(_sage(x_m, mm_edge_index, Wl_x1, bl_x1, Wr_x1, M))
    X1 = _ln(X1, g_x1, b_x1)
    X = jax.nn.relu(_sage(X1, mm_edge_index, Wl_x2, bl_x2, Wr_x2, M))
    X = _ln(X, g_x2, b_x2)
    Y1 = jax.nn.relu(_sage(x_d, dd_edge_index, Wl_y1, bl_y1, Wr_y1, D))
    Y1 = _ln(Y1, g_y1, b_y1)
    Y = jax.nn.relu(_sage(Y1, dd_edge_index, Wl_y2, bl_y2, Wr_y2, D))
    Y = _ln(Y, g_y2, b_y2)
    x = jax.nn.relu(X @ W1x.T + b1x)
    x = jax.nn.relu(x @ W2x.T + b2x)
    x = jax.nn.relu(x @ W3x.T + b3x)
    y = jax.nn.relu(Y @ W1y.T + b1y)
    y = jax.nn.relu(y @ W2y.T + b2y)
    y = jax.nn.relu(y @ W3y.T + b3y)
    return _final_matmul(x, y)


# speedup vs baseline: 1.7067x; 1.7067x over previous
import functools

import jax
import jax.numpy as jnp
from jax import lax
from jax.experimental import pallas as pl
from jax.experimental.pallas import tpu as pltpu
from jax.experimental.pallas import tpu_sc as plsc

N = 10000
NPAD = 10112
E = 320000
NW = 32
CW = 128
CHUNKS = 80
EPAD = NW * CHUNKS * CW
RB = 1000
SUBROWS = NPAD // 16


def _sc_segsum(table, src_r, dst_r, zeros, zerosv, with_cnt):
    mesh = plsc.VectorSubcoreMesh(core_axis_name="c", subcore_axis_name="s")
    out_type = [jax.ShapeDtypeStruct((2, NPAD, 128), jnp.float32)]
    if with_cnt:
        out_type.append(jax.ShapeDtypeStruct((NW, NPAD), jnp.float32))

    @functools.partial(
        pl.kernel,
        out_type=tuple(out_type),
        mesh=mesh,
        compiler_params=pltpu.CompilerParams(needs_layout_passes=False),
        scratch_types=[
            pltpu.VMEM((CHUNKS, CW), jnp.int32),
            pltpu.VMEM((CHUNKS, CW), jnp.int32),
            pltpu.VMEM((CW, 128), jnp.float32),
            pltpu.VMEM((NPAD,), jnp.float32),
            pltpu.VMEM_SHARED((NPAD, 128), jnp.float32),
            pltpu.SemaphoreType.DMA,
        ],
    )
    def k(table_hbm, src_hbm, dst_hbm, zeros_hbm, zerosv_hbm, *refs):
        if with_cnt:
            out_hbm, cnt_hbm = refs[0], refs[1]
            refs = refs[2:]
        else:
            out_hbm = refs[0]
            refs = refs[1:]
        src_v, dst_v, rows_v, cnt_v, acc_sh, sem = refs
        cid = lax.axis_index("c")
        sid = lax.axis_index("s")
        wid = sid * 2 + cid
        r0 = sid * SUBROWS
        pltpu.sync_copy(zeros_hbm.at[pl.ds(r0, SUBROWS)],
                        acc_sh.at[pl.ds(r0, SUBROWS)])
        if with_cnt:
            pltpu.sync_copy(zerosv_hbm, cnt_v)
        c0 = wid * CHUNKS
        pltpu.sync_copy(src_hbm.at[pl.ds(c0, CHUNKS)], src_v)
        pltpu.sync_copy(dst_hbm.at[pl.ds(c0, CHUNKS)], dst_v)
        plsc.subcore_barrier()
        ones16 = jnp.ones((16,), jnp.float32)

        def body(j, carry):
            pltpu.async_copy(table_hbm.at[src_v.at[j]], rows_v, sem).wait()
            pltpu.sync_copy(rows_v, acc_sh.at[dst_v.at[j]], add=True)
            if with_cnt:
                for v in range(CW // 16):
                    idx = dst_v[j, pl.ds(v * 16, 16)]
                    plsc.addupdate_scatter(cnt_v, [idx], ones16)
            return carry

        lax.fori_loop(0, CHUNKS, body, 0)
        plsc.subcore_barrier()
        pltpu.sync_copy(acc_sh.at[pl.ds(r0, SUBROWS)],
                        out_hbm.at[cid, pl.ds(r0, SUBROWS)])
        if with_cnt:
            pltpu.sync_copy(cnt_v, cnt_hbm.at[wid])

    return k(table, src_r, dst_r, zeros, zerosv)


def _prep_edges(edge_index):
    src = edge_index[0].astype(jnp.int32)
    dst = edge_index[1].astype(jnp.int32)
    pad = EPAD - E
    src_p = jnp.concatenate([src, jnp.zeros((pad,), jnp.int32)])
    dst_p = jnp.concatenate([dst, jnp.full((pad,), N, jnp.int32)])
    return src_p.reshape(NW * CHUNKS, CW), dst_p.reshape(NW * CHUNKS, CW)


def _dotT(a, w):
    return lax.dot_general(a, w, (((1,), (1,)), ((), ())),
                           preferred_element_type=jnp.float32)


def _layer1_block(p_ref, cp_ref, x_ref, wl_ref, bl_ref, wr_ref, g_ref, b_ref,
                  xh0_ref, xh1_ref, cnt_ref):
    agg = p_ref[0] + p_ref[1]
    cnt = jnp.sum(cp_ref[...], axis=-1)
    cntc = jnp.maximum(cnt, 1.0)
    mean = agg / cntc[:, None]
    h = _dotT(mean, wl_ref[...]) + _dotT(x_ref[...], wr_ref[...]) + bl_ref[...]
    h = jnp.maximum(h, 0.0)
    mu = jnp.mean(h, axis=1, keepdims=True)
    var = jnp.mean((h - mu) ** 2, axis=1, keepdims=True)
    hn = (h - mu) * lax.rsqrt(var + 1e-5) * g_ref[...] + b_ref[...]
    xh0_ref[...] = hn[:, :128]
    xh1_ref[...] = hn[:, 128:]
    cnt_ref[...] = jnp.broadcast_to(cntc[:, None], (RB, 8))


def _layer1(P, CPT, x, Wl, bl, Wr, g, b):
    return pl.pallas_call(
        _layer1_block,
        grid=(N // RB,),
        in_specs=[
            pl.BlockSpec((2, RB, 128), lambda i: (0, i, 0)),
            pl.BlockSpec((RB, NW), lambda i: (i, 0)),
            pl.BlockSpec((RB, 128), lambda i: (i, 0)),
            pl.BlockSpec((256, 128), lambda i: (0, 0)),
            pl.BlockSpec((256,), lambda i: (0,)),
            pl.BlockSpec((256, 128), lambda i: (0, 0)),
            pl.BlockSpec((256,), lambda i: (0,)),
            pl.BlockSpec((256,), lambda i: (0,)),
        ],
        out_specs=[
            pl.BlockSpec((RB, 128), lambda i: (i, 0)),
            pl.BlockSpec((RB, 128), lambda i: (i, 0)),
            pl.BlockSpec((RB, 8), lambda i: (i, 0)),
        ],
        out_shape=[
            jax.ShapeDtypeStruct((N, 128), jnp.float32),
            jax.ShapeDtypeStruct((N, 128), jnp.float32),
            jax.ShapeDtypeStruct((N, 8), jnp.float32),
        ],
    )(P, CPT, x, Wl, bl, Wr, g, b)


def _layer2_block(qa_ref, qb_ref, cnt_ref, xh0_ref, xh1_ref, wl_ref, bl_ref,
                  wr_ref, g_ref, b_ref, w1_ref, b1_ref, w2_ref, b2_ref,
                  w3_ref, b3_ref, o_ref):
    rinv = 1.0 / cnt_ref[:, :1]
    mean = jnp.concatenate(
        [(qa_ref[0] + qa_ref[1]) * rinv, (qb_ref[0] + qb_ref[1]) * rinv],
        axis=1)
    x1 = jnp.concatenate([xh0_ref[...], xh1_ref[...]], axis=1)
    h = _dotT(mean, wl_ref[...]) + _dotT(x1, wr_ref[...]) + bl_ref[...]
    h = jnp.maximum(h, 0.0)
    mu = jnp.mean(h, axis=1, keepdims=True)
    var = jnp.mean((h - mu) ** 2, axis=1, keepdims=True)
    hn = (h - mu) * lax.rsqrt(var + 1e-5) * g_ref[...] + b_ref[...]
    z = jnp.maximum(_dotT(hn, w1_ref[...]) + b1_ref[...], 0.0)
    z = jnp.maximum(_dotT(z, w2_ref[...]) + b2_ref[...], 0.0)
    z = jnp.maximum(_dotT(z, w3_ref[...]) + b3_ref[...], 0.0)
    o_ref[...] = z


def _layer2(Qa, Qb, cnt, Xh0, Xh1, Wl, bl, Wr, g, b, W1, b1, W2, b2, W3, b3):
    full = lambda r, c: pl.BlockSpec((r, c), lambda i: (0, 0))
    vec = lambda r: pl.BlockSpec((r,), lambda i: (0,))
    return pl.pallas_call(
        _layer2_block,
        grid=(N // RB,),
        in_specs=[
            pl.BlockSpec((2, RB, 128), lambda i: (0, i, 0)),
            pl.BlockSpec((2, RB, 128), lambda i: (0, i, 0)),
            pl.BlockSpec((RB, 8), lambda i: (i, 0)),
            pl.BlockSpec((RB, 128), lambda i: (i, 0)),
            pl.BlockSpec((RB, 128), lambda i: (i, 0)),
            full(128, 256), vec(128), full(128, 256), vec(128), vec(128),
            full(256, 128), vec(256), full(128, 256), vec(128),
            full(64, 128), vec(64),
        ],
        out_specs=pl.BlockSpec((RB, 64), lambda i: (i, 0)),
        out_shape=jax.ShapeDtypeStruct((N, 64), jnp.float32),
    )(Qa, Qb, cnt, Xh0, Xh1, Wl, bl, Wr, g, b, W1, b1, W2, b2, W3, b3)


def _final_block(x_ref, y_ref, o_ref):
    o_ref[...] = lax.dot_general(x_ref[...], y_ref[...],
                                 (((1,), (1,)), ((), ())),
                                 preferred_element_type=jnp.float32)


def _final_matmul(x, y):
    RBF = 400
    return pl.pallas_call(
        _final_block,
        grid=(N // RBF,),
        in_specs=[pl.BlockSpec((RBF, 64), lambda i: (i, 0)),
                  pl.BlockSpec((N, 64), lambda i: (0, 0))],
        out_specs=pl.BlockSpec((RBF, N), lambda i: (i, 0)),
        out_shape=jax.ShapeDtypeStruct((N, N), jnp.float32),
    )(x, y)


def _branch(x, edge_index, Wl1, bl1, Wr1, g1, b1, Wl2, bl2, Wr2, g2, b2,
            W1, c1, W2, c2, W3, c3):
    src_r, dst_r = _prep_edges(edge_index)
    zeros128 = jnp.zeros((NPAD, 128), jnp.float32)
    zerosv = jnp.zeros((NPAD,), jnp.float32)
    P, CNTP = _sc_segsum(x, src_r, dst_r, zeros128, zerosv, True)
    Xh0, Xh1, cnt = _layer1(P, CNTP.T, x, Wl1, bl1, Wr1, g1, b1)
    (Qa,) = _sc_segsum(Xh0, src_r, dst_r, zeros128, zerosv, False)
    (Qb,) = _sc_segsum(Xh1, src_r, dst_r, zeros128, zerosv, False)
    return _layer2(Qa, Qb, cnt, Xh0, Xh1, Wl2, bl2, Wr2, g2, b2,
                   W1, c1, W2, c2, W3, c3)


def kernel(mm_edge_index, dd_edge_index, x_m, x_d,
           Wl_x1, bl_x1, Wr_x1, g_x1, b_x1,
           Wl_x2, bl_x2, Wr_x2, g_x2, b_x2,
           Wl_y1, bl_y1, Wr_y1, g_y1, b_y1,
           Wl_y2, bl_y2, Wr_y2, g_y2, b_y2,
           W1x, b1x, W2x, b2x, W3x, b3x,
           W1y, b1y, W2y, b2y, W3y, b3y):
    xk = _branch(x_m, mm_edge_index, Wl_x1, bl_x1, Wr_x1, g_x1, b_x1,
                 Wl_x2, bl_x2, Wr_x2, g_x2, b_x2,
                 W1x, b1x, W2x, b2x, W3x, b3x)
    yk = _branch(x_d, dd_edge_index, Wl_y1, bl_y1, Wr_y1, g_y1, b_y1,
                 Wl_y2, bl_y2, Wr_y2, g_y2, b_y2,
                 W1y, b1y, W2y, b2y, W3y, b3y)
    return _final_matmul(xk, yk)


# speedup vs baseline: 1.8896x; 1.1072x over previous
import functools

import jax
import jax.numpy as jnp
from jax import lax
from jax.experimental import pallas as pl
from jax.experimental.pallas import tpu as pltpu
from jax.experimental.pallas import tpu_sc as plsc

N = 10000
NPAD = 10112
E = 320000
NW = 32
CW = 64
CHUNKS = 160
GCH = 1
EPAD = NW * CHUNKS * CW
RB = 1000
SUBROWS = NPAD // 16


def _sc_segsum(table, src_r, dst_r, zeros, zerosv, with_cnt):
    mesh = plsc.VectorSubcoreMesh(core_axis_name="c", subcore_axis_name="s")
    out_type = [jax.ShapeDtypeStruct((2, NPAD, 128), jnp.float32)]
    if with_cnt:
        out_type.append(jax.ShapeDtypeStruct((NW, NPAD), jnp.float32))

    @functools.partial(
        pl.kernel,
        out_type=tuple(out_type),
        mesh=mesh,
        compiler_params=pltpu.CompilerParams(needs_layout_passes=False),
        scratch_types=[
            pltpu.VMEM((CW,), jnp.int32),
            pltpu.VMEM((CW,), jnp.int32),
            pltpu.VMEM((CW,), jnp.int32),
            pltpu.VMEM((CW,), jnp.int32),
            pltpu.VMEM((CW, 128), jnp.float32),
            pltpu.VMEM((CW, 128), jnp.float32),
        ] + ([pltpu.VMEM((NPAD,), jnp.float32)] if with_cnt else []) + [
            pltpu.VMEM_SHARED((NPAD, 128), jnp.float32),
            pltpu.SemaphoreType.DMA,
            pltpu.SemaphoreType.DMA,
            pltpu.SemaphoreType.DMA,
            pltpu.SemaphoreType.DMA,
        ],
    )
    def k(table_hbm, src_hbm, dst_hbm, zeros_hbm, zerosv_hbm, *refs):
        if with_cnt:
            out_hbm, cnt_hbm = refs[0], refs[1]
            (isa, isb, ida, idb, rows_a, rows_b, cnt_v,
             acc_sh, gsa, gsb, ssa, ssb) = refs[2:]
        else:
            out_hbm = refs[0]
            cnt_v = None
            (isa, isb, ida, idb, rows_a, rows_b,
             acc_sh, gsa, gsb, ssa, ssb) = refs[1:]
        cid = lax.axis_index("c")
        sid = lax.axis_index("s")
        wid = sid * 2 + cid
        r0 = sid * SUBROWS
        pltpu.sync_copy(zeros_hbm.at[pl.ds(r0, SUBROWS)],
                        acc_sh.at[pl.ds(r0, SUBROWS)])
        if with_cnt:
            pltpu.sync_copy(zerosv_hbm, cnt_v)
        plsc.subcore_barrier()
        ones16 = jnp.ones((16,), jnp.float32)
        c0 = wid * CHUNKS

        def idx_load(is_v, id_v, g):
            pltpu.sync_copy(src_hbm.at[c0 + g], is_v)
            pltpu.sync_copy(dst_hbm.at[c0 + g], id_v)

        def g_issue(is_v, buf, sem):
            pltpu.async_copy(table_hbm.at[is_v], buf, sem)

        def g_drain(is_v, buf, sem):
            pltpu.make_async_copy(table_hbm.at[is_v], buf, sem).wait()

        def cnt_add(id_v):
            if with_cnt:
                for v in range(CW // 16):
                    plsc.addupdate_scatter(
                        cnt_v, [id_v[pl.ds(v * 16, 16)]], ones16)

        PAIRS = CHUNKS // 2

        idx_load(isa, ida, 0)
        g_issue(isa, rows_a, gsa)
        idx_load(isb, idb, 1)
        g_issue(isb, rows_b, gsb)

        def body(p, carry):
            a = 2 * p
            b = 2 * p + 1
            g_drain(isa, rows_a, gsa)
            d = pltpu.async_copy(rows_a, acc_sh.at[ida], ssa, add=True)
            cnt_add(ida)
            d.wait()

            @pl.when(a + 2 < CHUNKS)
            def _():
                idx_load(isa, ida, a + 2)
                g_issue(isa, rows_a, gsa)

            g_drain(isb, rows_b, gsb)
            d = pltpu.async_copy(rows_b, acc_sh.at[idb], ssb, add=True)
            cnt_add(idb)
            d.wait()

            @pl.when(b + 2 < CHUNKS)
            def _():
                idx_load(isb, idb, b + 2)
                g_issue(isb, rows_b, gsb)

            return carry

        lax.fori_loop(0, PAIRS, body, 0)
        plsc.subcore_barrier()
        pltpu.sync_copy(acc_sh.at[pl.ds(r0, SUBROWS)],
                        out_hbm.at[cid, pl.ds(r0, SUBROWS)])
        if with_cnt:
            pltpu.sync_copy(cnt_v, cnt_hbm.at[wid])

    return k(table, src_r, dst_r, zeros, zerosv)


def _prep_edges(edge_index):
    src = edge_index[0].astype(jnp.int32)
    dst = edge_index[1].astype(jnp.int32)
    pad = EPAD - E
    src_p = jnp.concatenate([src, jnp.zeros((pad,), jnp.int32)])
    dst_p = jnp.concatenate([dst, jnp.full((pad,), N, jnp.int32)])
    return src_p.reshape(NW * CHUNKS, CW), dst_p.reshape(NW * CHUNKS, CW)


def _dotT(a, w):
    return lax.dot_general(a, w, (((1,), (1,)), ((), ())),
                           preferred_element_type=jnp.float32)


def _layer1_block(p_ref, cp_ref, x_ref, wl_ref, bl_ref, wr_ref, g_ref, b_ref,
                  xh0_ref, xh1_ref, cnt_ref):
    agg = p_ref[0] + p_ref[1]
    cnt = jnp.sum(cp_ref[...], axis=-1)
    cntc = jnp.maximum(cnt, 1.0)
    mean = agg / cntc[:, None]
    h = _dotT(mean, wl_ref[...]) + _dotT(x_ref[...], wr_ref[...]) + bl_ref[...]
    h = jnp.maximum(h, 0.0)
    mu = jnp.mean(h, axis=1, keepdims=True)
    var = jnp.mean((h - mu) ** 2, axis=1, keepdims=True)
    hn = (h - mu) * lax.rsqrt(var + 1e-5) * g_ref[...] + b_ref[...]
    xh0_ref[...] = hn[:, :128]
    xh1_ref[...] = hn[:, 128:]
    cnt_ref[...] = jnp.broadcast_to(cntc[:, None], (RB, 8))


def _layer1(P, CPT, x, Wl, bl, Wr, g, b):
    return pl.pallas_call(
        _layer1_block,
        grid=(N // RB,),
        in_specs=[
            pl.BlockSpec((2, RB, 128), lambda i: (0, i, 0)),
            pl.BlockSpec((RB, NW), lambda i: (i, 0)),
            pl.BlockSpec((RB, 128), lambda i: (i, 0)),
            pl.BlockSpec((256, 128), lambda i: (0, 0)),
            pl.BlockSpec((256,), lambda i: (0,)),
            pl.BlockSpec((256, 128), lambda i: (0, 0)),
            pl.BlockSpec((256,), lambda i: (0,)),
            pl.BlockSpec((256,), lambda i: (0,)),
        ],
        out_specs=[
            pl.BlockSpec((RB, 128), lambda i: (i, 0)),
            pl.BlockSpec((RB, 128), lambda i: (i, 0)),
            pl.BlockSpec((RB, 8), lambda i: (i, 0)),
        ],
        out_shape=[
            jax.ShapeDtypeStruct((N, 128), jnp.float32),
            jax.ShapeDtypeStruct((N, 128), jnp.float32),
            jax.ShapeDtypeStruct((N, 8), jnp.float32),
        ],
    )(P, CPT, x, Wl, bl, Wr, g, b)


def _layer2_block(qa_ref, qb_ref, cnt_ref, xh0_ref, xh1_ref, wl_ref, bl_ref,
                  wr_ref, g_ref, b_ref, w1_ref, b1_ref, w2_ref, b2_ref,
                  w3_ref, b3_ref, o_ref):
    rinv = 1.0 / cnt_ref[:, :1]
    mean = jnp.concatenate(
        [(qa_ref[0] + qa_ref[1]) * rinv, (qb_ref[0] + qb_ref[1]) * rinv],
        axis=1)
    x1 = jnp.concatenate([xh0_ref[...], xh1_ref[...]], axis=1)
    h = _dotT(mean, wl_ref[...]) + _dotT(x1, wr_ref[...]) + bl_ref[...]
    h = jnp.maximum(h, 0.0)
    mu = jnp.mean(h, axis=1, keepdims=True)
    var = jnp.mean((h - mu) ** 2, axis=1, keepdims=True)
    hn = (h - mu) * lax.rsqrt(var + 1e-5) * g_ref[...] + b_ref[...]
    z = jnp.maximum(_dotT(hn, w1_ref[...]) + b1_ref[...], 0.0)
    z = jnp.maximum(_dotT(z, w2_ref[...]) + b2_ref[...], 0.0)
    z = jnp.maximum(_dotT(z, w3_ref[...]) + b3_ref[...], 0.0)
    o_ref[...] = z


def _layer2(Qa, Qb, cnt, Xh0, Xh1, Wl, bl, Wr, g, b, W1, b1, W2, b2, W3, b3):
    full = lambda r, c: pl.BlockSpec((r, c), lambda i: (0, 0))
    vec = lambda r: pl.BlockSpec((r,), lambda i: (0,))
    return pl.pallas_call(
        _layer2_block,
        grid=(N // RB,),
        in_specs=[
            pl.BlockSpec((2, RB, 128), lambda i: (0, i, 0)),
            pl.BlockSpec((2, RB, 128), lambda i: (0, i, 0)),
            pl.BlockSpec((RB, 8), lambda i: (i, 0)),
            pl.BlockSpec((RB, 128), lambda i: (i, 0)),
            pl.BlockSpec((RB, 128), lambda i: (i, 0)),
            full(128, 256), vec(128), full(128, 256), vec(128), vec(128),
            full(256, 128), vec(256), full(128, 256), vec(128),
            full(64, 128), vec(64),
        ],
        out_specs=pl.BlockSpec((RB, 64), lambda i: (i, 0)),
        out_shape=jax.ShapeDtypeStruct((N, 64), jnp.float32),
    )(Qa, Qb, cnt, Xh0, Xh1, Wl, bl, Wr, g, b, W1, b1, W2, b2, W3, b3)


def _final_block(x_ref, y_ref, o_ref):
    o_ref[...] = lax.dot_general(x_ref[...], y_ref[...],
                                 (((1,), (1,)), ((), ())),
                                 preferred_element_type=jnp.float32)


def _final_matmul(x, y):
    RBF = 400
    return pl.pallas_call(
        _final_block,
        grid=(N // RBF,),
        in_specs=[pl.BlockSpec((RBF, 64), lambda i: (i, 0)),
                  pl.BlockSpec((N, 64), lambda i: (0, 0))],
        out_specs=pl.BlockSpec((RBF, N), lambda i: (i, 0)),
        out_shape=jax.ShapeDtypeStruct((N, N), jnp.float32),
    )(x, y)


def _branch(x, edge_index, Wl1, bl1, Wr1, g1, b1, Wl2, bl2, Wr2, g2, b2,
            W1, c1, W2, c2, W3, c3):
    src_r, dst_r = _prep_edges(edge_index)
    zeros128 = jnp.zeros((NPAD, 128), jnp.float32)
    zerosv = jnp.zeros((NPAD,), jnp.float32)
    P, CNTP = _sc_segsum(x, src_r, dst_r, zeros128, zerosv, True)
    Xh0, Xh1, cnt = _layer1(P, CNTP.T, x, Wl1, bl1, Wr1, g1, b1)
    (Qa,) = _sc_segsum(Xh0, src_r, dst_r, zeros128, zerosv, False)
    (Qb,) = _sc_segsum(Xh1, src_r, dst_r, zeros128, zerosv, False)
    return _layer2(Qa, Qb, cnt, Xh0, Xh1, Wl2, bl2, Wr2, g2, b2,
                   W1, c1, W2, c2, W3, c3)


def kernel(mm_edge_index, dd_edge_index, x_m, x_d,
           Wl_x1, bl_x1, Wr_x1, g_x1, b_x1,
           Wl_x2, bl_x2, Wr_x2, g_x2, b_x2,
           Wl_y1, bl_y1, Wr_y1, g_y1, b_y1,
           Wl_y2, bl_y2, Wr_y2, g_y2, b_y2,
           W1x, b1x, W2x, b2x, W3x, b3x,
           W1y, b1y, W2y, b2y, W3y, b3y):
    xk = _branch(x_m, mm_edge_index, Wl_x1, bl_x1, Wr_x1, g_x1, b_x1,
                 Wl_x2, bl_x2, Wr_x2, g_x2, b_x2,
                 W1x, b1x, W2x, b2x, W3x, b3x)
    yk = _branch(x_d, dd_edge_index, Wl_y1, bl_y1, Wr_y1, g_y1, b_y1,
                 Wl_y2, bl_y2, Wr_y2, g_y2, b_y2,
                 W1y, b1y, W2y, b2y, W3y, b3y)
    return _final_matmul(xk, yk)
